# Initial kernel scaffold; baseline (speedup 1.0000x reference)
import jax
import jax.numpy as jnp
from jax.experimental import pallas as pl


def kernel(x, node_keep_mask, source, target, edge_type, bases, relation_base_weights):
    raise NotImplementedError("write your pallas kernel here")


# speedup vs baseline: 49.3366x; 49.3366x over previous
import functools

import jax
import jax.numpy as jnp
from jax import lax
from jax.experimental import pallas as pl
from jax.experimental.pallas import tpu as pltpu
from jax.experimental.pallas import tpu_sc as plsc

N_NODES = 10000
N_EDGES = 320000
D = 128
R = 8
B = 4

NC = 2
NS = 16
L = 16
NW = NC * NS

C = 80
EPT = N_EDGES // NW
NCHUNK = EPT // C
G = 25
NGROUP = NCHUNK // G
NPAD = 10240
RPT = NPAD // NS

BN = 1000
NBLK = N_NODES // BN


def _phase1_body(x_ref, bases_ref, rbw_ref, maskf_ref, m_ref, self_ref):
    x = x_ref[...]
    ys = [
        jnp.dot(x, bases_ref[b], preferred_element_type=jnp.float32)
        for b in range(B)
    ]
    for r in range(R):
        m = ys[0] * rbw_ref[r, 0]
        for b in range(1, B):
            m = m + ys[b] * rbw_ref[r, b]
        m_ref[r] = m
    s = ys[0] * rbw_ref[R, 0]
    for b in range(1, B):
        s = s + ys[b] * rbw_ref[R, b]
    self_ref[...] = s * maskf_ref[...]


def _phase1(x, bases, rbw, maskf):
    return pl.pallas_call(
        _phase1_body,
        grid=(NBLK,),
        in_specs=[
            pl.BlockSpec((BN, D), lambda j: (j, 0)),
            pl.BlockSpec((B, D, D), lambda j: (0, 0, 0)),
            pl.BlockSpec(memory_space=pltpu.SMEM),
            pl.BlockSpec((BN, 1), lambda j: (j, 0)),
        ],
        out_specs=[
            pl.BlockSpec((R, BN, D), lambda j: (0, j, 0)),
            pl.BlockSpec((BN, D), lambda j: (j, 0)),
        ],
        out_shape=[
            jax.ShapeDtypeStruct((R, N_NODES, D), jnp.float32),
            jax.ShapeDtypeStruct((N_NODES, D), jnp.float32),
        ],
    )(x, bases, rbw, maskf)


def _sc_body(src2_hbm, tgt2_hbm, et2_hbm, m_hbm, zeros_hbm, out_hbm,
             srcb, tgtb, etb, g1, g2, rows1, rows2, acc):
    cid = lax.axis_index("c")
    sid = lax.axis_index("s")
    wid = sid * NC + cid

    pltpu.sync_copy(zeros_hbm, acc.at[pl.ds(sid * RPT, RPT), :])
    plsc.subcore_barrier()

    def group_body(g, _):
        pltpu.sync_copy(src2_hbm.at[wid, g], srcb)
        pltpu.sync_copy(tgt2_hbm.at[wid, g], tgtb)
        pltpu.sync_copy(et2_hbm.at[wid, g], etb)

        def vec_body(i, _):
            j = i // (C // L)
            k = (i % (C // L)) * L
            et16 = etb[j, pl.ds(k, L)] * N_NODES
            g1[j, pl.ds(k, L)] = et16 + srcb[j, pl.ds(k, L)]
            g2[j, pl.ds(k, L)] = et16 + tgtb[j, pl.ds(k, L)]
            return 0

        lax.fori_loop(0, G * (C // L), vec_body, 0)

        def chunk_body(j, _):
            pltpu.sync_copy(m_hbm.at[g1.at[j]], rows1)
            pltpu.sync_copy(rows1, acc.at[tgtb.at[j]], add=True)
            pltpu.sync_copy(m_hbm.at[g2.at[j]], rows2)
            pltpu.sync_copy(rows2, acc.at[srcb.at[j]], add=True)
            return 0

        lax.fori_loop(0, G, chunk_body, 0)
        return 0

    lax.fori_loop(0, NGROUP, group_body, 0)
    plsc.subcore_barrier()

    pltpu.sync_copy(acc.at[pl.ds(sid * RPT, RPT), :],
                    out_hbm.at[cid, pl.ds(sid * RPT, RPT), :])


@functools.cache
def _sc_scatter():
    return functools.partial(
        pl.kernel,
        out_type=jax.ShapeDtypeStruct((NC, NPAD, D), jnp.float32),
        mesh=plsc.VectorSubcoreMesh(core_axis_name="c", subcore_axis_name="s"),
        scratch_types=[
            pltpu.VMEM((G, C), jnp.int32),
            pltpu.VMEM((G, C), jnp.int32),
            pltpu.VMEM((G, C), jnp.int32),
            pltpu.VMEM((G, C), jnp.int32),
            pltpu.VMEM((G, C), jnp.int32),
            pltpu.VMEM((C, D), jnp.float32),
            pltpu.VMEM((C, D), jnp.float32),
            pltpu.VMEM_SHARED((NPAD, D), jnp.float32),
        ],
    )(_sc_body)


def _phase3_body(self_ref, p_ref, o_ref):
    o_ref[...] = self_ref[...] + p_ref[0] + p_ref[1]


def _phase3(self_part, p):
    return pl.pallas_call(
        _phase3_body,
        grid=(NBLK,),
        in_specs=[
            pl.BlockSpec((BN, D), lambda j: (j, 0)),
            pl.BlockSpec((NC, BN, D), lambda j: (0, j, 0)),
        ],
        out_specs=pl.BlockSpec((BN, D), lambda j: (j, 0)),
        out_shape=jax.ShapeDtypeStruct((N_NODES, D), jnp.float32),
    )(self_part, p)


def kernel(x, node_keep_mask, source, target, edge_type, bases,
           relation_base_weights):
    maskf = node_keep_mask.astype(jnp.float32).reshape(N_NODES, 1)
    m, self_part = _phase1(x, bases, relation_base_weights, maskf)
    m2 = m.reshape(R * N_NODES, D)
    src2 = source.reshape(NW, NGROUP, G, C)
    tgt2 = target.reshape(NW, NGROUP, G, C)
    et2 = edge_type.reshape(NW, NGROUP, G, C)
    zeros = jnp.zeros((RPT, D), jnp.float32)
    p = _sc_scatter()(src2, tgt2, et2, m2, zeros)
    return _phase3(self_part, p)


# speedup vs baseline: 63.3459x; 1.2840x over previous
import functools

import jax
import jax.numpy as jnp
from jax import lax
from jax.experimental import pallas as pl
from jax.experimental.pallas import tpu as pltpu
from jax.experimental.pallas import tpu_sc as plsc

N_NODES = 10000
N_EDGES = 320000
D = 128
R = 8
B = 4

NC = 2
NS = 16
L = 16
NW = NC * NS

C = 80
EPT = N_EDGES // NW
NCHUNK = EPT // C
G = 25
NGROUP = NCHUNK // G
NPAD = 10240
RPT = NPAD // NS

BN = 1000
NBLK = N_NODES // BN


def _phase1_body(x_ref, bases_ref, rbw_ref, maskf_ref, m_ref, self_ref):
    x = x_ref[...]
    ys = [
        jnp.dot(x, bases_ref[b], preferred_element_type=jnp.float32)
        for b in range(B)
    ]
    for r in range(R):
        m = ys[0] * rbw_ref[r, 0]
        for b in range(1, B):
            m = m + ys[b] * rbw_ref[r, b]
        m_ref[r] = m
    s = ys[0] * rbw_ref[R, 0]
    for b in range(1, B):
        s = s + ys[b] * rbw_ref[R, b]
    self_ref[...] = s * maskf_ref[...]


def _phase1(x, bases, rbw, maskf):
    return pl.pallas_call(
        _phase1_body,
        grid=(NBLK,),
        in_specs=[
            pl.BlockSpec((BN, D), lambda j: (j, 0)),
            pl.BlockSpec((B, D, D), lambda j: (0, 0, 0)),
            pl.BlockSpec(memory_space=pltpu.SMEM),
            pl.BlockSpec((BN, 1), lambda j: (j, 0)),
        ],
        out_specs=[
            pl.BlockSpec((R, BN, D), lambda j: (0, j, 0)),
            pl.BlockSpec((BN, D), lambda j: (j, 0)),
        ],
        out_shape=[
            jax.ShapeDtypeStruct((R, N_NODES, D), jnp.float32),
            jax.ShapeDtypeStruct((N_NODES, D), jnp.float32),
        ],
    )(x, bases, rbw, maskf)


def _sc_body(src2_hbm, tgt2_hbm, et2_hbm, m_hbm, zeros_hbm, out_hbm,
             srcb, tgtb, etb, g1, g2, rows_a, rows_b, acc,
             sga, sgb, ssa, ssb):
    cid = lax.axis_index("c")
    sid = lax.axis_index("s")
    wid = sid * NC + cid

    pltpu.sync_copy(zeros_hbm, acc.at[pl.ds(sid * RPT, RPT), :])
    plsc.subcore_barrier()

    def group_body(g, _):
        pltpu.sync_copy(src2_hbm.at[wid, g], srcb)
        pltpu.sync_copy(tgt2_hbm.at[wid, g], tgtb)
        pltpu.sync_copy(et2_hbm.at[wid, g], etb)

        def vec_body(i, _):
            j = i // (C // L)
            k = (i % (C // L)) * L
            et16 = etb[j, pl.ds(k, L)] * N_NODES
            g1[j, pl.ds(k, L)] = et16 + srcb[j, pl.ds(k, L)]
            g2[j, pl.ds(k, L)] = et16 + tgtb[j, pl.ds(k, L)]
            return 0

        lax.fori_loop(0, G * (C // L), vec_body, 0)

        def chunk_body(j, _):
            @pl.when(j > 0)
            def _():
                pltpu.make_async_copy(rows_a, acc.at[tgtb.at[j - 1]], ssa).wait()

            pltpu.async_copy(m_hbm.at[g1.at[j]], rows_a, sga)

            @pl.when(j > 0)
            def _():
                pltpu.make_async_copy(rows_b, acc.at[srcb.at[j - 1]], ssb).wait()

            pltpu.async_copy(m_hbm.at[g2.at[j]], rows_b, sgb)

            pltpu.make_async_copy(m_hbm.at[g1.at[j]], rows_a, sga).wait()
            pltpu.async_copy(rows_a, acc.at[tgtb.at[j]], ssa, add=True)
            pltpu.make_async_copy(m_hbm.at[g2.at[j]], rows_b, sgb).wait()
            pltpu.async_copy(rows_b, acc.at[srcb.at[j]], ssb, add=True)
            return 0

        lax.fori_loop(0, G, chunk_body, 0)
        pltpu.make_async_copy(rows_a, acc.at[tgtb.at[G - 1]], ssa).wait()
        pltpu.make_async_copy(rows_b, acc.at[srcb.at[G - 1]], ssb).wait()
        return 0

    lax.fori_loop(0, NGROUP, group_body, 0)
    plsc.subcore_barrier()

    pltpu.sync_copy(acc.at[pl.ds(sid * RPT, RPT), :],
                    out_hbm.at[cid, pl.ds(sid * RPT, RPT), :])


@functools.cache
def _sc_scatter():
    return functools.partial(
        pl.kernel,
        out_type=jax.ShapeDtypeStruct((NC, NPAD, D), jnp.float32),
        mesh=plsc.VectorSubcoreMesh(core_axis_name="c", subcore_axis_name="s"),
        scratch_types=[
            pltpu.VMEM((G, C), jnp.int32),
            pltpu.VMEM((G, C), jnp.int32),
            pltpu.VMEM((G, C), jnp.int32),
            pltpu.VMEM((G, C), jnp.int32),
            pltpu.VMEM((G, C), jnp.int32),
            pltpu.VMEM((C, D), jnp.float32),
            pltpu.VMEM((C, D), jnp.float32),
            pltpu.VMEM_SHARED((NPAD, D), jnp.float32),
            pltpu.SemaphoreType.DMA,
            pltpu.SemaphoreType.DMA,
            pltpu.SemaphoreType.DMA,
            pltpu.SemaphoreType.DMA,
        ],
    )(_sc_body)


def _phase3_body(self_ref, p_ref, o_ref):
    o_ref[...] = self_ref[...] + p_ref[0] + p_ref[1]


def _phase3(self_part, p):
    return pl.pallas_call(
        _phase3_body,
        grid=(NBLK,),
        in_specs=[
            pl.BlockSpec((BN, D), lambda j: (j, 0)),
            pl.BlockSpec((NC, BN, D), lambda j: (0, j, 0)),
        ],
        out_specs=pl.BlockSpec((BN, D), lambda j: (j, 0)),
        out_shape=jax.ShapeDtypeStruct((N_NODES, D), jnp.float32),
    )(self_part, p)


def kernel(x, node_keep_mask, source, target, edge_type, bases,
           relation_base_weights):
    maskf = node_keep_mask.astype(jnp.float32).reshape(N_NODES, 1)
    m, self_part = _phase1(x, bases, relation_base_weights, maskf)
    m2 = m.reshape(R * N_NODES, D)
    src2 = source.reshape(NW, NGROUP, G, C)
    tgt2 = target.reshape(NW, NGROUP, G, C)
    et2 = edge_type.reshape(NW, NGROUP, G, C)
    zeros = jnp.zeros((RPT, D), jnp.float32)
    p = _sc_scatter()(src2, tgt2, et2, m2, zeros)
    return _phase3(self_part, p)


# speedup vs baseline: 90.5590x; 1.4296x over previous
import functools

import jax
import jax.numpy as jnp
from jax import lax
from jax.experimental import pallas as pl
from jax.experimental.pallas import tpu as pltpu
from jax.experimental.pallas import tpu_sc as plsc

N_NODES = 10000
N_EDGES = 320000
D = 128
R = 8
B = 4

NC = 2
NS = 16
L = 16
NW = NC * NS

C = 80
EPT = N_EDGES // NW
NCHUNK = EPT // C
G = 25
NGROUP = NCHUNK // G
RPT = 632
RLAST = N_NODES - 15 * RPT

BN = 1000
NBLK = N_NODES // BN


def _phase1_body(x_ref, bases_ref, rbw_ref, maskf_ref, m_ref, self_ref):
    x = x_ref[...]
    ys = [
        jnp.dot(x, bases_ref[b], preferred_element_type=jnp.float32)
        for b in range(B)
    ]
    for r in range(R):
        m = ys[0] * rbw_ref[r, 0]
        for b in range(1, B):
            m = m + ys[b] * rbw_ref[r, b]
        m_ref[r] = m
    s = ys[0] * rbw_ref[R, 0]
    for b in range(1, B):
        s = s + ys[b] * rbw_ref[R, b]
    self_ref[...] = s * maskf_ref[...]


def _phase1(x, bases, rbw, maskf):
    return pl.pallas_call(
        _phase1_body,
        grid=(NBLK,),
        in_specs=[
            pl.BlockSpec((BN, D), lambda j: (j, 0)),
            pl.BlockSpec((B, D, D), lambda j: (0, 0, 0)),
            pl.BlockSpec(memory_space=pltpu.SMEM),
            pl.BlockSpec((BN, 1), lambda j: (j, 0)),
        ],
        out_specs=[
            pl.BlockSpec((R, BN, D), lambda j: (0, j, 0)),
            pl.BlockSpec((BN, D), lambda j: (j, 0)),
        ],
        out_shape=[
            jax.ShapeDtypeStruct((R, N_NODES, D), jnp.float32),
            jax.ShapeDtypeStruct((N_NODES, D), jnp.float32),
        ],
    )(x, bases, rbw, maskf)


def _sc_body(src2_hbm, tgt2_hbm, et2_hbm, m_hbm, zeros_hbm, out_hbm,
             idxall, rowsall, acc, sg0, sg1, sg2, ss0, ss1, ss2):
    cid = lax.axis_index("c")
    sid = lax.axis_index("s")
    wid = sid * NC + cid

    rbufs = (rowsall.at[pl.ds(0, C), :],
             rowsall.at[pl.ds(C, C), :],
             rowsall.at[pl.ds(2 * C, C), :])
    sgs = (sg0, sg1, sg2)
    sss = (ss0, ss1, ss2)

    srcb = idxall.at[pl.ds(0, G), :]
    tgtb = idxall.at[pl.ds(G, G), :]
    g1b = idxall.at[pl.ds(2 * G, G), :]
    g2b = idxall.at[pl.ds(3 * G, G), :]

    def gath(j, rows, sem, gidx):
        pltpu.async_copy(m_hbm.at[gidx.at[j]], rows, sem)

    def gath_wait(j, rows, sem, gidx):
        pltpu.make_async_copy(m_hbm.at[gidx.at[j]], rows, sem).wait()

    def scat(j, rows, sem, sidx):
        pltpu.async_copy(rows, acc.at[sidx.at[j]], sem, add=True)

    def scat_wait(j, rows, sem, sidx):
        pltpu.make_async_copy(rows, acc.at[sidx.at[j]], sem).wait()

    @pl.when(sid < NS - 1)
    def _():
        pltpu.sync_copy(zeros_hbm, acc.at[pl.ds(sid * RPT, RPT), :])

    @pl.when(sid == NS - 1)
    def _():
        pltpu.sync_copy(zeros_hbm.at[pl.ds(0, RLAST), :],
                        acc.at[pl.ds(sid * RPT, RLAST), :])

    plsc.subcore_barrier()

    def group_body(g, _):
        pltpu.sync_copy(src2_hbm.at[wid, g], idxall.at[pl.ds(0, G), :])
        pltpu.sync_copy(tgt2_hbm.at[wid, g], idxall.at[pl.ds(G, G), :])
        pltpu.sync_copy(et2_hbm.at[wid, g], idxall.at[pl.ds(2 * G, G), :])

        def vec_body(i, _):
            j = i // (C // L)
            k = (i % (C // L)) * L
            et16 = idxall[2 * G + j, pl.ds(k, L)] * N_NODES
            idxall[3 * G + j, pl.ds(k, L)] = et16 + idxall[G + j, pl.ds(k, L)]
            idxall[2 * G + j, pl.ds(k, L)] = et16 + idxall[j, pl.ds(k, L)]
            return 0

        lax.fori_loop(0, G * (C // L), vec_body, 0)

        bands_g = (g1b, g2b)
        bands_s = (tgtb, srcb)

        def slot(t, p, guarded):
            rows_b = rbufs[p % 3]

            def step_a():
                scat_wait(3 * t + (p - 3) // 2, rows_b,
                          sss[p % 3], bands_s[(p - 3) % 2])

            def step_c():
                rows_c = rbufs[(p - 2) % 3]
                gath_wait(3 * t + (p - 2) // 2, rows_c,
                          sgs[(p - 2) % 3], bands_g[p % 2])
                scat(3 * t + (p - 2) // 2, rows_c,
                     sss[(p - 2) % 3], bands_s[p % 2])

            if p >= 3 or not guarded:
                step_a()
            else:
                pl.when(t > 0)(step_a)
            gath(3 * t + p // 2, rows_b, sgs[p % 3], bands_g[p % 2])
            if p >= 2 or not guarded:
                step_c()
            else:
                pl.when(t > 0)(step_c)

        def ring_body(t, _):
            for p in range(6):
                slot(t, p, guarded=True)
            return 0

        lax.fori_loop(0, 8, ring_body, 0)

        scat_wait(22, rbufs[0], sss[0], srcb)
        gath(G - 1, rbufs[0], sgs[0], g1b)
        gath_wait(23, rbufs[1], sgs[1], g1b)
        scat(23, rbufs[1], sss[1], tgtb)
        scat_wait(23, rbufs[1], sss[1], tgtb)
        gath(G - 1, rbufs[1], sgs[1], g2b)
        gath_wait(23, rbufs[2], sgs[2], g2b)
        scat(23, rbufs[2], sss[2], srcb)
        gath_wait(G - 1, rbufs[0], sgs[0], g1b)
        scat(G - 1, rbufs[0], sss[0], tgtb)
        gath_wait(G - 1, rbufs[1], sgs[1], g2b)
        scat(G - 1, rbufs[1], sss[1], srcb)
        scat_wait(23, rbufs[2], sss[2], srcb)
        scat_wait(G - 1, rbufs[0], sss[0], tgtb)
        scat_wait(G - 1, rbufs[1], sss[1], srcb)
        return 0

    lax.fori_loop(0, NGROUP, group_body, 0)
    plsc.subcore_barrier()

    @pl.when(sid < NS - 1)
    def _():
        pltpu.sync_copy(acc.at[pl.ds(sid * RPT, RPT), :],
                        out_hbm.at[cid, pl.ds(sid * RPT, RPT), :])

    @pl.when(sid == NS - 1)
    def _():
        pltpu.sync_copy(acc.at[pl.ds(sid * RPT, RLAST), :],
                        out_hbm.at[cid, pl.ds(sid * RPT, RLAST), :])


@functools.cache
def _sc_scatter():
    return functools.partial(
        pl.kernel,
        out_type=jax.ShapeDtypeStruct((NC, N_NODES, D), jnp.float32),
        mesh=plsc.VectorSubcoreMesh(core_axis_name="c", subcore_axis_name="s"),
        scratch_types=[
            pltpu.VMEM((4 * G, C), jnp.int32),
            pltpu.VMEM((3 * C, D), jnp.float32),
            pltpu.VMEM_SHARED((N_NODES, D), jnp.float32),
            pltpu.SemaphoreType.DMA,
            pltpu.SemaphoreType.DMA,
            pltpu.SemaphoreType.DMA,
            pltpu.SemaphoreType.DMA,
            pltpu.SemaphoreType.DMA,
            pltpu.SemaphoreType.DMA,
        ],
    )(_sc_body)


def _phase3_body(self_ref, p_ref, o_ref):
    o_ref[...] = self_ref[...] + p_ref[0] + p_ref[1]


def _phase3(self_part, p):
    return pl.pallas_call(
        _phase3_body,
        grid=(NBLK,),
        in_specs=[
            pl.BlockSpec((BN, D), lambda j: (j, 0)),
            pl.BlockSpec((NC, BN, D), lambda j: (0, j, 0)),
        ],
        out_specs=pl.BlockSpec((BN, D), lambda j: (j, 0)),
        out_shape=jax.ShapeDtypeStruct((N_NODES, D), jnp.float32),
    )(self_part, p)


def kernel(x, node_keep_mask, source, target, edge_type, bases,
           relation_base_weights):
    maskf = node_keep_mask.astype(jnp.float32).reshape(N_NODES, 1)
    m, self_part = _phase1(x, bases, relation_base_weights, maskf)
    m2 = m.reshape(R * N_NODES, D)
    src2 = source.reshape(NW, NGROUP, G, C)
    tgt2 = target.reshape(NW, NGROUP, G, C)
    et2 = edge_type.reshape(NW, NGROUP, G, C)
    zeros = jnp.zeros((RPT, D), jnp.float32)
    p = _sc_scatter()(src2, tgt2, et2, m2, zeros)
    return _phase3(self_part, p)


# speedup vs baseline: 90.7127x; 1.0017x over previous
import functools

import jax
import jax.numpy as jnp
from jax import lax
from jax.experimental import pallas as pl
from jax.experimental.pallas import tpu as pltpu
from jax.experimental.pallas import tpu_sc as plsc

N_NODES = 10000
N_EDGES = 320000
D = 128
R = 8
B = 4

NC = 2
NS = 16
L = 16
NW = NC * NS

C = 80
EPT = N_EDGES // NW
NCHUNK = EPT // C
G = 25
NGROUP = NCHUNK // G
RPT = 632
RLAST = N_NODES - 15 * RPT

BN = 1000
NBLK = N_NODES // BN


def _phase1_body(x_ref, bases_ref, rbw_ref, maskf_ref, m_ref, self_ref):
    x = x_ref[...]
    ys = [
        jnp.dot(x, bases_ref[b], preferred_element_type=jnp.float32)
        for b in range(B)
    ]
    for r in range(R):
        m = ys[0] * rbw_ref[r, 0]
        for b in range(1, B):
            m = m + ys[b] * rbw_ref[r, b]
        m_ref[r] = m
    s = ys[0] * rbw_ref[R, 0]
    for b in range(1, B):
        s = s + ys[b] * rbw_ref[R, b]
    self_ref[...] = s * maskf_ref[...]


def _phase1(x, bases, rbw, maskf):
    return pl.pallas_call(
        _phase1_body,
        grid=(NBLK,),
        in_specs=[
            pl.BlockSpec((BN, D), lambda j: (j, 0)),
            pl.BlockSpec((B, D, D), lambda j: (0, 0, 0)),
            pl.BlockSpec(memory_space=pltpu.SMEM),
            pl.BlockSpec((BN, 1), lambda j: (j, 0)),
        ],
        out_specs=[
            pl.BlockSpec((R, BN, D), lambda j: (0, j, 0)),
            pl.BlockSpec((BN, D), lambda j: (j, 0)),
        ],
        out_shape=[
            jax.ShapeDtypeStruct((R, N_NODES, D), jnp.float32),
            jax.ShapeDtypeStruct((N_NODES, D), jnp.float32),
        ],
    )(x, bases, rbw, maskf)


def _sc_body(src2_hbm, tgt2_hbm, et2_hbm, m_hbm, zeros_hbm, out_hbm,
             idxall, rowsall, acc, sg0, sg1, sg2, ss0, ss1, ss2):
    cid = lax.axis_index("c")
    sid = lax.axis_index("s")
    wid = sid * NC + cid

    rbufs = (rowsall.at[pl.ds(0, C), :],
             rowsall.at[pl.ds(C, C), :],
             rowsall.at[pl.ds(2 * C, C), :])
    sgs = (sg0, sg1, sg2)
    sss = (ss0, ss1, ss2)

    srcb = idxall.at[pl.ds(0, G), :]
    tgtb = idxall.at[pl.ds(G, G), :]
    g1b = idxall.at[pl.ds(2 * G, G), :]
    g2b = idxall.at[pl.ds(3 * G, G), :]

    H = C // 2

    def gath(j, rows, sem, gidx):
        pltpu.async_copy(m_hbm.at[gidx.at[j, pl.ds(0, H)]],
                         rows.at[pl.ds(0, H), :], sem)
        pltpu.async_copy(m_hbm.at[gidx.at[j, pl.ds(H, H)]],
                         rows.at[pl.ds(H, H), :], sem)

    def gath_wait(j, rows, sem, gidx):
        pltpu.make_async_copy(m_hbm.at[gidx.at[j, pl.ds(0, H)]],
                              rows.at[pl.ds(0, H), :], sem).wait()
        pltpu.make_async_copy(m_hbm.at[gidx.at[j, pl.ds(H, H)]],
                              rows.at[pl.ds(H, H), :], sem).wait()

    def scat(j, rows, sem, sidx):
        pltpu.async_copy(rows, acc.at[sidx.at[j]], sem, add=True)

    def scat_wait(j, rows, sem, sidx):
        pltpu.make_async_copy(rows, acc.at[sidx.at[j]], sem).wait()

    @pl.when(sid < NS - 1)
    def _():
        pltpu.sync_copy(zeros_hbm, acc.at[pl.ds(sid * RPT, RPT), :])

    @pl.when(sid == NS - 1)
    def _():
        pltpu.sync_copy(zeros_hbm.at[pl.ds(0, RLAST), :],
                        acc.at[pl.ds(sid * RPT, RLAST), :])

    plsc.subcore_barrier()

    def group_body(g, _):
        pltpu.sync_copy(src2_hbm.at[wid, g], idxall.at[pl.ds(0, G), :])
        pltpu.sync_copy(tgt2_hbm.at[wid, g], idxall.at[pl.ds(G, G), :])
        pltpu.sync_copy(et2_hbm.at[wid, g], idxall.at[pl.ds(2 * G, G), :])

        def vec_body(i, _):
            j = i // (C // L)
            k = (i % (C // L)) * L
            et16 = idxall[2 * G + j, pl.ds(k, L)] * N_NODES
            idxall[3 * G + j, pl.ds(k, L)] = et16 + idxall[G + j, pl.ds(k, L)]
            idxall[2 * G + j, pl.ds(k, L)] = et16 + idxall[j, pl.ds(k, L)]
            return 0

        lax.fori_loop(0, G * (C // L), vec_body, 0)

        bands_g = (g1b, g2b)
        bands_s = (tgtb, srcb)

        def slot(t, p, guarded):
            rows_b = rbufs[p % 3]

            def step_a():
                scat_wait(3 * t + (p - 3) // 2, rows_b,
                          sss[p % 3], bands_s[(p - 3) % 2])

            def step_c():
                rows_c = rbufs[(p - 2) % 3]
                gath_wait(3 * t + (p - 2) // 2, rows_c,
                          sgs[(p - 2) % 3], bands_g[p % 2])
                scat(3 * t + (p - 2) // 2, rows_c,
                     sss[(p - 2) % 3], bands_s[p % 2])

            if p >= 3 or not guarded:
                step_a()
            else:
                pl.when(t > 0)(step_a)
            gath(3 * t + p // 2, rows_b, sgs[p % 3], bands_g[p % 2])
            if p >= 2 or not guarded:
                step_c()
            else:
                pl.when(t > 0)(step_c)

        def ring_body(t, _):
            for p in range(6):
                slot(t, p, guarded=True)
            return 0

        lax.fori_loop(0, 8, ring_body, 0)

        scat_wait(22, rbufs[0], sss[0], srcb)
        gath(G - 1, rbufs[0], sgs[0], g1b)
        gath_wait(23, rbufs[1], sgs[1], g1b)
        scat(23, rbufs[1], sss[1], tgtb)
        scat_wait(23, rbufs[1], sss[1], tgtb)
        gath(G - 1, rbufs[1], sgs[1], g2b)
        gath_wait(23, rbufs[2], sgs[2], g2b)
        scat(23, rbufs[2], sss[2], srcb)
        gath_wait(G - 1, rbufs[0], sgs[0], g1b)
        scat(G - 1, rbufs[0], sss[0], tgtb)
        gath_wait(G - 1, rbufs[1], sgs[1], g2b)
        scat(G - 1, rbufs[1], sss[1], srcb)
        scat_wait(23, rbufs[2], sss[2], srcb)
        scat_wait(G - 1, rbufs[0], sss[0], tgtb)
        scat_wait(G - 1, rbufs[1], sss[1], srcb)
        return 0

    lax.fori_loop(0, NGROUP, group_body, 0)
    plsc.subcore_barrier()

    @pl.when(sid < NS - 1)
    def _():
        pltpu.sync_copy(acc.at[pl.ds(sid * RPT, RPT), :],
                        out_hbm.at[cid, pl.ds(sid * RPT, RPT), :])

    @pl.when(sid == NS - 1)
    def _():
        pltpu.sync_copy(acc.at[pl.ds(sid * RPT, RLAST), :],
                        out_hbm.at[cid, pl.ds(sid * RPT, RLAST), :])


@functools.cache
def _sc_scatter():
    return functools.partial(
        pl.kernel,
        out_type=jax.ShapeDtypeStruct((NC, N_NODES, D), jnp.float32),
        mesh=plsc.VectorSubcoreMesh(core_axis_name="c", subcore_axis_name="s"),
        scratch_types=[
            pltpu.VMEM((4 * G, C), jnp.int32),
            pltpu.VMEM((3 * C, D), jnp.float32),
            pltpu.VMEM_SHARED((N_NODES, D), jnp.float32),
            pltpu.SemaphoreType.DMA,
            pltpu.SemaphoreType.DMA,
            pltpu.SemaphoreType.DMA,
            pltpu.SemaphoreType.DMA,
            pltpu.SemaphoreType.DMA,
            pltpu.SemaphoreType.DMA,
        ],
    )(_sc_body)


def _phase3_body(self_ref, p_ref, o_ref):
    o_ref[...] = self_ref[...] + p_ref[0] + p_ref[1]


def _phase3(self_part, p):
    return pl.pallas_call(
        _phase3_body,
        grid=(NBLK,),
        in_specs=[
            pl.BlockSpec((BN, D), lambda j: (j, 0)),
            pl.BlockSpec((NC, BN, D), lambda j: (0, j, 0)),
        ],
        out_specs=pl.BlockSpec((BN, D), lambda j: (j, 0)),
        out_shape=jax.ShapeDtypeStruct((N_NODES, D), jnp.float32),
    )(self_part, p)


def kernel(x, node_keep_mask, source, target, edge_type, bases,
           relation_base_weights):
    maskf = node_keep_mask.astype(jnp.float32).reshape(N_NODES, 1)
    m, self_part = _phase1(x, bases, relation_base_weights, maskf)
    m2 = m.reshape(R * N_NODES, D)
    src2 = source.reshape(NW, NGROUP, G, C)
    tgt2 = target.reshape(NW, NGROUP, G, C)
    et2 = edge_type.reshape(NW, NGROUP, G, C)
    zeros = jnp.zeros((RPT, D), jnp.float32)
    p = _sc_scatter()(src2, tgt2, et2, m2, zeros)
    return _phase3(self_part, p)


# speedup vs baseline: 92.7768x; 1.0228x over previous
import functools

import jax
import jax.numpy as jnp
from jax import lax
from jax.experimental import pallas as pl
from jax.experimental.pallas import tpu as pltpu
from jax.experimental.pallas import tpu_sc as plsc

N_NODES = 10000
N_EDGES = 320000
D = 128
R = 8
B = 4

NC = 2
NS = 16
L = 16
NW = NC * NS

C = 80
EPT = N_EDGES // NW
NCHUNK = EPT // C
G = 25
NGROUP = NCHUNK // G
RPT = 632
RLAST = N_NODES - 15 * RPT

BN = 1000
NBLK = N_NODES // BN


def _phase1_body(x_ref, bases_ref, rbw_ref, m_ref):
    x = x_ref[...]
    ys = [
        jnp.dot(x, bases_ref[b], preferred_element_type=jnp.float32)
        for b in range(B)
    ]
    for r in range(R):
        m = ys[0] * rbw_ref[r, 0]
        for b in range(1, B):
            m = m + ys[b] * rbw_ref[r, b]
        m_ref[r] = m


def _phase1(x, bases, rbw):
    return pl.pallas_call(
        _phase1_body,
        grid=(NBLK,),
        in_specs=[
            pl.BlockSpec((BN, D), lambda j: (j, 0)),
            pl.BlockSpec((B, D, D), lambda j: (0, 0, 0)),
            pl.BlockSpec(memory_space=pltpu.SMEM),
        ],
        out_specs=pl.BlockSpec((R, BN, D), lambda j: (0, j, 0)),
        out_shape=jax.ShapeDtypeStruct((R, N_NODES, D), jnp.float32),
    )(x, bases, rbw)


def _sc_body(src2_hbm, tgt2_hbm, et2_hbm, m_hbm, zeros_hbm, out_hbm,
             idxall, rowsall, acc, sg0, sg1, sg2, ss0, ss1, ss2):
    cid = lax.axis_index("c")
    sid = lax.axis_index("s")
    wid = sid * NC + cid

    rbufs = (rowsall.at[pl.ds(0, C), :],
             rowsall.at[pl.ds(C, C), :],
             rowsall.at[pl.ds(2 * C, C), :])
    sgs = (sg0, sg1, sg2)
    sss = (ss0, ss1, ss2)

    srcb = idxall.at[pl.ds(0, G), :]
    tgtb = idxall.at[pl.ds(G, G), :]
    g1b = idxall.at[pl.ds(2 * G, G), :]
    g2b = idxall.at[pl.ds(3 * G, G), :]

    H = C // 2

    def gath(j, rows, sem, gidx):
        pltpu.async_copy(m_hbm.at[gidx.at[j, pl.ds(0, H)]],
                         rows.at[pl.ds(0, H), :], sem)
        pltpu.async_copy(m_hbm.at[gidx.at[j, pl.ds(H, H)]],
                         rows.at[pl.ds(H, H), :], sem)

    def gath_wait(j, rows, sem, gidx):
        pltpu.make_async_copy(m_hbm.at[gidx.at[j, pl.ds(0, H)]],
                              rows.at[pl.ds(0, H), :], sem).wait()
        pltpu.make_async_copy(m_hbm.at[gidx.at[j, pl.ds(H, H)]],
                              rows.at[pl.ds(H, H), :], sem).wait()

    def scat(j, rows, sem, sidx):
        pltpu.async_copy(rows, acc.at[sidx.at[j]], sem, add=True)

    def scat_wait(j, rows, sem, sidx):
        pltpu.make_async_copy(rows, acc.at[sidx.at[j]], sem).wait()

    @pl.when(sid < NS - 1)
    def _():
        pltpu.sync_copy(zeros_hbm, acc.at[pl.ds(sid * RPT, RPT), :])

    @pl.when(sid == NS - 1)
    def _():
        pltpu.sync_copy(zeros_hbm.at[pl.ds(0, RLAST), :],
                        acc.at[pl.ds(sid * RPT, RLAST), :])

    plsc.subcore_barrier()

    def group_body(g, _):
        pltpu.sync_copy(src2_hbm.at[wid, g], idxall.at[pl.ds(0, G), :])
        pltpu.sync_copy(tgt2_hbm.at[wid, g], idxall.at[pl.ds(G, G), :])
        pltpu.sync_copy(et2_hbm.at[wid, g], idxall.at[pl.ds(2 * G, G), :])

        def vec_body(i, _):
            j = i // (C // L)
            k = (i % (C // L)) * L
            et16 = idxall[2 * G + j, pl.ds(k, L)] * N_NODES
            idxall[3 * G + j, pl.ds(k, L)] = et16 + idxall[G + j, pl.ds(k, L)]
            idxall[2 * G + j, pl.ds(k, L)] = et16 + idxall[j, pl.ds(k, L)]
            return 0

        lax.fori_loop(0, G * (C // L), vec_body, 0)

        bands_g = (g1b, g2b)
        bands_s = (tgtb, srcb)

        def slot(t, p, guarded):
            rows_b = rbufs[p % 3]

            def step_a():
                scat_wait(3 * t + (p - 3) // 2, rows_b,
                          sss[p % 3], bands_s[(p - 3) % 2])

            def step_c():
                rows_c = rbufs[(p - 2) % 3]
                gath_wait(3 * t + (p - 2) // 2, rows_c,
                          sgs[(p - 2) % 3], bands_g[p % 2])
                scat(3 * t + (p - 2) // 2, rows_c,
                     sss[(p - 2) % 3], bands_s[p % 2])

            if p >= 3 or not guarded:
                step_a()
            else:
                pl.when(t > 0)(step_a)
            gath(3 * t + p // 2, rows_b, sgs[p % 3], bands_g[p % 2])
            if p >= 2 or not guarded:
                step_c()
            else:
                pl.when(t > 0)(step_c)

        def ring_body(t, _):
            for p in range(6):
                slot(t, p, guarded=True)
            return 0

        lax.fori_loop(0, 8, ring_body, 0)

        scat_wait(22, rbufs[0], sss[0], srcb)
        gath(G - 1, rbufs[0], sgs[0], g1b)
        gath_wait(23, rbufs[1], sgs[1], g1b)
        scat(23, rbufs[1], sss[1], tgtb)
        scat_wait(23, rbufs[1], sss[1], tgtb)
        gath(G - 1, rbufs[1], sgs[1], g2b)
        gath_wait(23, rbufs[2], sgs[2], g2b)
        scat(23, rbufs[2], sss[2], srcb)
        gath_wait(G - 1, rbufs[0], sgs[0], g1b)
        scat(G - 1, rbufs[0], sss[0], tgtb)
        gath_wait(G - 1, rbufs[1], sgs[1], g2b)
        scat(G - 1, rbufs[1], sss[1], srcb)
        scat_wait(23, rbufs[2], sss[2], srcb)
        scat_wait(G - 1, rbufs[0], sss[0], tgtb)
        scat_wait(G - 1, rbufs[1], sss[1], srcb)
        return 0

    lax.fori_loop(0, NGROUP, group_body, 0)
    plsc.subcore_barrier()

    @pl.when(sid < NS - 1)
    def _():
        pltpu.sync_copy(acc.at[pl.ds(sid * RPT, RPT), :],
                        out_hbm.at[cid, pl.ds(sid * RPT, RPT), :])

    @pl.when(sid == NS - 1)
    def _():
        pltpu.sync_copy(acc.at[pl.ds(sid * RPT, RLAST), :],
                        out_hbm.at[cid, pl.ds(sid * RPT, RLAST), :])


@functools.cache
def _sc_scatter():
    return functools.partial(
        pl.kernel,
        out_type=jax.ShapeDtypeStruct((NC, N_NODES, D), jnp.float32),
        mesh=plsc.VectorSubcoreMesh(core_axis_name="c", subcore_axis_name="s"),
        scratch_types=[
            pltpu.VMEM((4 * G, C), jnp.int32),
            pltpu.VMEM((3 * C, D), jnp.float32),
            pltpu.VMEM_SHARED((N_NODES, D), jnp.float32),
            pltpu.SemaphoreType.DMA,
            pltpu.SemaphoreType.DMA,
            pltpu.SemaphoreType.DMA,
            pltpu.SemaphoreType.DMA,
            pltpu.SemaphoreType.DMA,
            pltpu.SemaphoreType.DMA,
        ],
    )(_sc_body)


def _phase3_body(x_ref, bases_ref, rbw_ref, maskf_ref, p_ref, o_ref):
    w = bases_ref[0] * rbw_ref[R, 0]
    for b in range(1, B):
        w = w + bases_ref[b] * rbw_ref[R, b]
    s = jnp.dot(x_ref[...], w, preferred_element_type=jnp.float32)
    o_ref[...] = s * maskf_ref[...] + p_ref[0] + p_ref[1]


def _phase3(x, bases, rbw, maskf, p):
    return pl.pallas_call(
        _phase3_body,
        grid=(NBLK,),
        in_specs=[
            pl.BlockSpec((BN, D), lambda j: (j, 0)),
            pl.BlockSpec((B, D, D), lambda j: (0, 0, 0)),
            pl.BlockSpec(memory_space=pltpu.SMEM),
            pl.BlockSpec((BN, 1), lambda j: (j, 0)),
            pl.BlockSpec((NC, BN, D), lambda j: (0, j, 0)),
        ],
        out_specs=pl.BlockSpec((BN, D), lambda j: (j, 0)),
        out_shape=jax.ShapeDtypeStruct((N_NODES, D), jnp.float32),
    )(x, bases, rbw, maskf, p)


def kernel(x, node_keep_mask, source, target, edge_type, bases,
           relation_base_weights):
    maskf = node_keep_mask.astype(jnp.float32).reshape(N_NODES, 1)
    m = _phase1(x, bases, relation_base_weights)
    m2 = m.reshape(R * N_NODES, D)
    src2 = source.reshape(NW, NGROUP, G, C)
    tgt2 = target.reshape(NW, NGROUP, G, C)
    et2 = edge_type.reshape(NW, NGROUP, G, C)
    zeros = jnp.zeros((RPT, D), jnp.float32)
    p = _sc_scatter()(src2, tgt2, et2, m2, zeros)
    return _phase3(x, bases, relation_base_weights, maskf, p)


# speedup vs baseline: 94.0519x; 1.0137x over previous
import functools

import jax
import jax.numpy as jnp
from jax import lax
from jax.experimental import pallas as pl
from jax.experimental.pallas import tpu as pltpu
from jax.experimental.pallas import tpu_sc as plsc

N_NODES = 10000
N_EDGES = 320000
D = 128
R = 8
B = 4

NC = 2
NS = 16
L = 16
NW = NC * NS

C = 80
EPT = N_EDGES // NW
NCHUNK = EPT // C
G = 25
NGROUP = NCHUNK // G
RPT = 632
RLAST = N_NODES - 15 * RPT

BN = 2000
NBLK = N_NODES // BN


def _phase1_body(x_ref, bases_ref, rbw_ref, m_ref):
    x = x_ref[...]
    ys = [
        jnp.dot(x, bases_ref[b], preferred_element_type=jnp.float32)
        for b in range(B)
    ]
    for r in range(R):
        m = ys[0] * rbw_ref[r, 0]
        for b in range(1, B):
            m = m + ys[b] * rbw_ref[r, b]
        m_ref[r] = m


def _phase1(x, bases, rbw):
    return pl.pallas_call(
        _phase1_body,
        grid=(NBLK,),
        in_specs=[
            pl.BlockSpec((BN, D), lambda j: (j, 0)),
            pl.BlockSpec((B, D, D), lambda j: (0, 0, 0)),
            pl.BlockSpec(memory_space=pltpu.SMEM),
        ],
        out_specs=pl.BlockSpec((R, BN, D), lambda j: (0, j, 0)),
        out_shape=jax.ShapeDtypeStruct((R, N_NODES, D), jnp.float32),
    )(x, bases, rbw)


def _sc_body(src2_hbm, tgt2_hbm, et2_hbm, m_hbm, zeros_hbm, out_hbm,
             idxall, rowsall, acc, sg0, sg1, sg2, ss0, ss1, ss2):
    cid = lax.axis_index("c")
    sid = lax.axis_index("s")
    wid = sid * NC + cid

    rbufs = (rowsall.at[pl.ds(0, C), :],
             rowsall.at[pl.ds(C, C), :],
             rowsall.at[pl.ds(2 * C, C), :])
    sgs = (sg0, sg1, sg2)
    sss = (ss0, ss1, ss2)

    srcb = idxall.at[pl.ds(0, G), :]
    tgtb = idxall.at[pl.ds(G, G), :]
    g1b = idxall.at[pl.ds(2 * G, G), :]
    g2b = idxall.at[pl.ds(3 * G, G), :]

    H = C // 2

    def gath(j, rows, sem, gidx):
        pltpu.async_copy(m_hbm.at[gidx.at[j, pl.ds(0, H)]],
                         rows.at[pl.ds(0, H), :], sem)
        pltpu.async_copy(m_hbm.at[gidx.at[j, pl.ds(H, H)]],
                         rows.at[pl.ds(H, H), :], sem)

    def gath_wait(j, rows, sem, gidx):
        pltpu.make_async_copy(m_hbm.at[gidx.at[j, pl.ds(0, H)]],
                              rows.at[pl.ds(0, H), :], sem).wait()
        pltpu.make_async_copy(m_hbm.at[gidx.at[j, pl.ds(H, H)]],
                              rows.at[pl.ds(H, H), :], sem).wait()

    def scat(j, rows, sem, sidx):
        pltpu.async_copy(rows, acc.at[sidx.at[j]], sem, add=True)

    def scat_wait(j, rows, sem, sidx):
        pltpu.make_async_copy(rows, acc.at[sidx.at[j]], sem).wait()

    @pl.when(sid < NS - 1)
    def _():
        pltpu.sync_copy(zeros_hbm, acc.at[pl.ds(sid * RPT, RPT), :])

    @pl.when(sid == NS - 1)
    def _():
        pltpu.sync_copy(zeros_hbm.at[pl.ds(0, RLAST), :],
                        acc.at[pl.ds(sid * RPT, RLAST), :])

    plsc.subcore_barrier()

    def group_body(g, _):
        pltpu.sync_copy(src2_hbm.at[wid, g], idxall.at[pl.ds(0, G), :])
        pltpu.sync_copy(tgt2_hbm.at[wid, g], idxall.at[pl.ds(G, G), :])
        pltpu.sync_copy(et2_hbm.at[wid, g], idxall.at[pl.ds(2 * G, G), :])

        def vec_body(i, _):
            j = i // (C // L)
            k = (i % (C // L)) * L
            et16 = idxall[2 * G + j, pl.ds(k, L)] * N_NODES
            idxall[3 * G + j, pl.ds(k, L)] = et16 + idxall[G + j, pl.ds(k, L)]
            idxall[2 * G + j, pl.ds(k, L)] = et16 + idxall[j, pl.ds(k, L)]
            return 0

        lax.fori_loop(0, G * (C // L), vec_body, 0)

        bands_g = (g1b, g2b)
        bands_s = (tgtb, srcb)

        def slot(t, p, guarded):
            rows_b = rbufs[p % 3]

            def step_a():
                scat_wait(3 * t + (p - 3) // 2, rows_b,
                          sss[p % 3], bands_s[(p - 3) % 2])

            def step_c():
                rows_c = rbufs[(p - 2) % 3]
                gath_wait(3 * t + (p - 2) // 2, rows_c,
                          sgs[(p - 2) % 3], bands_g[p % 2])
                scat(3 * t + (p - 2) // 2, rows_c,
                     sss[(p - 2) % 3], bands_s[p % 2])

            if p >= 3 or not guarded:
                step_a()
            else:
                pl.when(t > 0)(step_a)
            gath(3 * t + p // 2, rows_b, sgs[p % 3], bands_g[p % 2])
            if p >= 2 or not guarded:
                step_c()
            else:
                pl.when(t > 0)(step_c)

        def ring_body(t, _):
            for p in range(6):
                slot(t, p, guarded=True)
            return 0

        lax.fori_loop(0, 8, ring_body, 0)

        scat_wait(22, rbufs[0], sss[0], srcb)
        gath(G - 1, rbufs[0], sgs[0], g1b)
        gath_wait(23, rbufs[1], sgs[1], g1b)
        scat(23, rbufs[1], sss[1], tgtb)
        scat_wait(23, rbufs[1], sss[1], tgtb)
        gath(G - 1, rbufs[1], sgs[1], g2b)
        gath_wait(23, rbufs[2], sgs[2], g2b)
        scat(23, rbufs[2], sss[2], srcb)
        gath_wait(G - 1, rbufs[0], sgs[0], g1b)
        scat(G - 1, rbufs[0], sss[0], tgtb)
        gath_wait(G - 1, rbufs[1], sgs[1], g2b)
        scat(G - 1, rbufs[1], sss[1], srcb)
        scat_wait(23, rbufs[2], sss[2], srcb)
        scat_wait(G - 1, rbufs[0], sss[0], tgtb)
        scat_wait(G - 1, rbufs[1], sss[1], srcb)
        return 0

    lax.fori_loop(0, NGROUP, group_body, 0)
    plsc.subcore_barrier()

    @pl.when(sid < NS - 1)
    def _():
        pltpu.sync_copy(acc.at[pl.ds(sid * RPT, RPT), :],
                        out_hbm.at[cid, pl.ds(sid * RPT, RPT), :])

    @pl.when(sid == NS - 1)
    def _():
        pltpu.sync_copy(acc.at[pl.ds(sid * RPT, RLAST), :],
                        out_hbm.at[cid, pl.ds(sid * RPT, RLAST), :])


@functools.cache
def _sc_scatter():
    return functools.partial(
        pl.kernel,
        out_type=jax.ShapeDtypeStruct((NC, N_NODES, D), jnp.float32),
        mesh=plsc.VectorSubcoreMesh(core_axis_name="c", subcore_axis_name="s"),
        scratch_types=[
            pltpu.VMEM((4 * G, C), jnp.int32),
            pltpu.VMEM((3 * C, D), jnp.float32),
            pltpu.VMEM_SHARED((N_NODES, D), jnp.float32),
            pltpu.SemaphoreType.DMA,
            pltpu.SemaphoreType.DMA,
            pltpu.SemaphoreType.DMA,
            pltpu.SemaphoreType.DMA,
            pltpu.SemaphoreType.DMA,
            pltpu.SemaphoreType.DMA,
        ],
    )(_sc_body)


def _phase3_body(x_ref, bases_ref, rbw_ref, maskf_ref, p_ref, o_ref):
    w = bases_ref[0] * rbw_ref[R, 0]
    for b in range(1, B):
        w = w + bases_ref[b] * rbw_ref[R, b]
    s = jnp.dot(x_ref[...], w, preferred_element_type=jnp.float32)
    o_ref[...] = s * maskf_ref[...] + p_ref[0] + p_ref[1]


def _phase3(x, bases, rbw, maskf, p):
    return pl.pallas_call(
        _phase3_body,
        grid=(NBLK,),
        in_specs=[
            pl.BlockSpec((BN, D), lambda j: (j, 0)),
            pl.BlockSpec((B, D, D), lambda j: (0, 0, 0)),
            pl.BlockSpec(memory_space=pltpu.SMEM),
            pl.BlockSpec((BN, 1), lambda j: (j, 0)),
            pl.BlockSpec((NC, BN, D), lambda j: (0, j, 0)),
        ],
        out_specs=pl.BlockSpec((BN, D), lambda j: (j, 0)),
        out_shape=jax.ShapeDtypeStruct((N_NODES, D), jnp.float32),
    )(x, bases, rbw, maskf, p)


def kernel(x, node_keep_mask, source, target, edge_type, bases,
           relation_base_weights):
    maskf = node_keep_mask.astype(jnp.float32).reshape(N_NODES, 1)
    m = _phase1(x, bases, relation_base_weights)
    m2 = m.reshape(R * N_NODES, D)
    src2 = source.reshape(NW, NGROUP, G, C)
    tgt2 = target.reshape(NW, NGROUP, G, C)
    et2 = edge_type.reshape(NW, NGROUP, G, C)
    zeros = jnp.zeros((RPT, D), jnp.float32)
    p = _sc_scatter()(src2, tgt2, et2, m2, zeros)
    return _phase3(x, bases, relation_base_weights, maskf, p)
